# Initial kernel scaffold; baseline (speedup 1.0000x reference)
import jax
import jax.numpy as jnp
from jax.experimental import pallas as pl


def kernel(x, edge_index, edge_type, Wl, bl, W1, b1, W2, b2, W3, b3, Wu, bu, Ws, bs):
    raise NotImplementedError("write your pallas kernel here")


# speedup vs baseline: 19.6090x; 19.6090x over previous
import functools

import jax
import jax.numpy as jnp
from jax import lax
from jax.experimental import pallas as pl
from jax.experimental.pallas import tpu as pltpu
from jax.experimental.pallas import tpu_sc as plsc

NPAD = 10240
BLK = 1024
CH = 128
NC = 2
NS = 16
NW = NC * NS


def _lower_body(x_ref, w_ref, b_ref, o_ref):
    o_ref[...] = jnp.maximum(
        jnp.dot(x_ref[...], w_ref[...], preferred_element_type=jnp.float32)
        + b_ref[...], 0.0)


def _lower(x, W, b):
    n, h = x.shape
    d = W.shape[1]
    return pl.pallas_call(
        _lower_body,
        grid=(n // BLK,),
        in_specs=[
            pl.BlockSpec((BLK, h), lambda i: (i, 0)),
            pl.BlockSpec((h, d), lambda i: (0, 0)),
            pl.BlockSpec((1, d), lambda i: (0, 0)),
        ],
        out_specs=pl.BlockSpec((BLK, d), lambda i: (i, 0)),
        out_shape=jax.ShapeDtypeStruct((n, d), jnp.float32),
    )(x, W, b.reshape(1, -1))


def _table_body(h_ref, w_ref, o_ref):
    o_ref[...] = jnp.dot(h_ref[...], w_ref[0],
                         preferred_element_type=jnp.float32)[None]


def _table(h, W):
    n, d = h.shape
    r = W.shape[0]
    return pl.pallas_call(
        _table_body,
        grid=(n // BLK, r),
        in_specs=[
            pl.BlockSpec((BLK, d), lambda i, j: (i, 0)),
            pl.BlockSpec((1, d, d), lambda i, j: (j, 0, 0)),
        ],
        out_specs=pl.BlockSpec((1, BLK, d), lambda i, j: (j, i, 0)),
        out_shape=jax.ShapeDtypeStruct((r, n, d), jnp.float32),
    )(h, W)


def _combine_body(a_ref, b_ref, o_ref):
    o_ref[...] = jnp.maximum(a_ref[0] + a_ref[1] + b_ref[...], 0.0)


def _combine(agg, b):
    n, d = agg.shape[1], agg.shape[2]
    return pl.pallas_call(
        _combine_body,
        grid=(n // BLK,),
        in_specs=[
            pl.BlockSpec((2, BLK, d), lambda i: (0, i, 0)),
            pl.BlockSpec((1, d), lambda i: (0, 0)),
        ],
        out_specs=pl.BlockSpec((BLK, d), lambda i: (i, 0)),
        out_shape=jax.ShapeDtypeStruct((n, d), jnp.float32),
    )(agg, b.reshape(1, -1))


def _final_body(h_ref, wu_ref, bu_ref, x_ref, ws_ref, bs_ref, o_ref):
    up = jnp.maximum(
        jnp.dot(h_ref[...], wu_ref[...], preferred_element_type=jnp.float32)
        + bu_ref[...], 0.0)
    sk = jnp.maximum(
        jnp.dot(x_ref[...], ws_ref[...], preferred_element_type=jnp.float32)
        + bs_ref[...], 0.0)
    o_ref[...] = up + sk


def _final(h, Wu, bu, x, Ws, bs):
    n, d = h.shape
    hd = x.shape[1]
    return pl.pallas_call(
        _final_body,
        grid=(n // BLK,),
        in_specs=[
            pl.BlockSpec((BLK, d), lambda i: (i, 0)),
            pl.BlockSpec((d, hd), lambda i: (0, 0)),
            pl.BlockSpec((1, hd), lambda i: (0, 0)),
            pl.BlockSpec((BLK, hd), lambda i: (i, 0)),
            pl.BlockSpec((hd, hd), lambda i: (0, 0)),
            pl.BlockSpec((1, hd), lambda i: (0, 0)),
        ],
        out_specs=pl.BlockSpec((BLK, hd), lambda i: (i, 0)),
        out_shape=jax.ShapeDtypeStruct((n, hd), jnp.float32),
    )(h, Wu, bu.reshape(1, -1), x, Ws, bs.reshape(1, -1))


def _edge_pass(table, gidx, dst, zeros):
    e = gidx.shape[0]
    d = table.shape[1]
    nchunks = e // CH
    base_ch, extra = nchunks // NW, nchunks % NW
    rps = NPAD // NS
    mesh = plsc.VectorSubcoreMesh(core_axis_name="c", subcore_axis_name="s")

    @functools.partial(
        pl.kernel,
        out_type=jax.ShapeDtypeStruct((NC, NPAD, d), jnp.float32),
        mesh=mesh,
        scratch_types=[
            pltpu.VMEM((CH,), jnp.int32),
            pltpu.VMEM((CH,), jnp.int32),
            pltpu.VMEM((CH, d), jnp.float32),
            pltpu.VMEM_SHARED((NPAD, d), jnp.float32),
            pltpu.SemaphoreType.DMA,
        ],
    )
    def body(table_hbm, gidx_hbm, dst_hbm, zeros_hbm, out_hbm,
             gidx_v, dst_v, rows_v, acc, sem):
        c = lax.axis_index("c")
        s = lax.axis_index("s")
        w = s * NC + c
        r0 = s * rps
        pltpu.sync_copy(zeros_hbm.at[pl.ds(r0, rps)], acc.at[pl.ds(r0, rps)])
        plsc.subcore_barrier()
        nch = base_ch + jnp.where(w < extra, 1, 0)

        def chunk(i, carry):
            start = (i * NW + w) * CH
            pltpu.sync_copy(gidx_hbm.at[pl.ds(start, CH)], gidx_v)
            pltpu.sync_copy(dst_hbm.at[pl.ds(start, CH)], dst_v)
            pltpu.async_copy(table_hbm.at[gidx_v], rows_v, sem).wait()
            pltpu.sync_copy(rows_v, acc.at[dst_v], add=True)
            return carry

        lax.fori_loop(0, nch, chunk, 0)
        plsc.subcore_barrier()
        pltpu.sync_copy(acc.at[pl.ds(r0, rps)],
                        out_hbm.at[c, pl.ds(r0, rps)])

    return body(table, gidx, dst, zeros)


def kernel(x, edge_index, edge_type, Wl, bl, W1, b1, W2, b2, W3, b3,
           Wu, bu, Ws, bs):
    n = x.shape[0]
    d = Wl.shape[1]
    src = edge_index[0]
    dst = edge_index[1]
    gidx = edge_type * NPAD + src
    zeros = jnp.zeros((NPAD, d), jnp.float32)
    xp = jnp.pad(x, ((0, NPAD - n), (0, 0)))

    h = _lower(xp, Wl, bl)
    for W, b in ((W1, b1), (W2, b2), (W3, b3)):
        t = _table(h, W).reshape(-1, d)
        agg = _edge_pass(t, gidx, dst, zeros)
        h = _combine(agg, b)
    out = _final(h, Wu, bu, xp, Ws, bs)
    return out[:n]
